# baseline (device time: 19983 ns/iter reference)
import jax
import jax.numpy as jnp
from jax import lax
from jax.experimental import pallas as pl
from jax.experimental.pallas import tpu as pltpu

N_DEV = 4
N_HALF = 4


def kernel(A, B):
    m, k = A.shape
    _, n = B.shape
    ch = m // N_DEV
    nh = n // N_HALF

    def body(a_ref, b_ref, out_ref, partial_ref, rs_buf,
             rs_send, rs_recv, ag_send, ag_recv):
        me = lax.axis_index("i")

        bar = pltpu.get_barrier_semaphore()
        for d in range(1, N_DEV):
            pl.semaphore_signal(
                bar, inc=1,
                device_id=((me + d) % N_DEV,),
                device_id_type=pl.DeviceIdType.MESH,
            )
        pl.semaphore_wait(bar, N_DEV - 1)

        rs = {}
        for h in range(N_HALF):
            partial_ref[:, pl.ds(h * nh, nh)] = jnp.dot(
                a_ref[:, :], b_ref[:, pl.ds(h * nh, nh)],
                preferred_element_type=jnp.float32,
            )
            for d in (2, 1, 3):
                tgt = (me + d) % N_DEV
                rdma = pltpu.make_async_remote_copy(
                    src_ref=partial_ref.at[pl.ds(tgt * ch, ch),
                                           pl.ds(h * nh, nh)],
                    dst_ref=rs_buf.at[h, d - 1],
                    send_sem=rs_send.at[h, d - 1],
                    recv_sem=rs_recv.at[h, d - 1],
                    device_id=(tgt,),
                    device_id_type=pl.DeviceIdType.MESH,
                )
                rdma.start()
                rs[(h, d)] = rdma

        ag = []
        for h in range(N_HALF):
            for d in (1, 3, 2):
                rs[(h, d)].wait_recv()
            out_ref[pl.ds(me * ch, ch), pl.ds(h * nh, nh)] = (
                partial_ref[pl.ds(me * ch, ch), pl.ds(h * nh, nh)]
                + rs_buf[h, 0] + rs_buf[h, 1] + rs_buf[h, 2]
            )
            for d in (2, 1, 3):
                tgt = (me + d) % N_DEV
                rdma = pltpu.make_async_remote_copy(
                    src_ref=out_ref.at[pl.ds(me * ch, ch),
                                       pl.ds(h * nh, nh)],
                    dst_ref=out_ref.at[pl.ds(me * ch, ch),
                                       pl.ds(h * nh, nh)],
                    send_sem=ag_send.at[h, d - 1],
                    recv_sem=ag_recv.at[h, d - 1],
                    device_id=(tgt,),
                    device_id_type=pl.DeviceIdType.MESH,
                )
                rdma.start()
                ag.append(rdma)

        for r in rs.values():
            r.wait_send()
        for r in ag:
            r.wait()

    return pl.pallas_call(
        body,
        out_shape=jax.ShapeDtypeStruct((m, n), jnp.float32),
        in_specs=[
            pl.BlockSpec(memory_space=pltpu.VMEM),
            pl.BlockSpec(memory_space=pltpu.VMEM),
        ],
        out_specs=pl.BlockSpec(memory_space=pltpu.VMEM),
        scratch_shapes=[
            pltpu.VMEM((m, n), jnp.float32),
            pltpu.VMEM((N_HALF, N_DEV - 1, ch, nh), jnp.float32),
            pltpu.SemaphoreType.DMA((N_HALF, N_DEV - 1)),
            pltpu.SemaphoreType.DMA((N_HALF, N_DEV - 1)),
            pltpu.SemaphoreType.DMA((N_HALF, N_DEV - 1)),
            pltpu.SemaphoreType.DMA((N_HALF, N_DEV - 1)),
        ],
        compiler_params=pltpu.CompilerParams(collective_id=0),
    )(A, B)


# device time: 19796 ns/iter; 1.0094x vs baseline; 1.0094x over previous
import jax
import jax.numpy as jnp
from jax import lax
from jax.experimental import pallas as pl
from jax.experimental.pallas import tpu as pltpu

N_DEV = 4
N_HALF = 2


def kernel(A, B):
    m, k = A.shape
    _, n = B.shape
    ch = m // N_DEV
    nh = n // N_HALF

    def body(a_ref, b_ref, out_ref, partial_ref, rs_buf, acc_ref,
             rs_send, rs_recv, ag_send, ag_recv, local_sem):
        me = lax.axis_index("i")

        bar = pltpu.get_barrier_semaphore()
        for d in range(1, N_DEV):
            pl.semaphore_signal(
                bar, inc=1,
                device_id=((me + d) % N_DEV,),
                device_id_type=pl.DeviceIdType.MESH,
            )
        pl.semaphore_wait(bar, N_DEV - 1)

        rs = {}
        for h in range(N_HALF):
            partial_ref[:, pl.ds(h * nh, nh)] = jnp.dot(
                a_ref[:, :], b_ref[:, pl.ds(h * nh, nh)],
                preferred_element_type=jnp.float32,
            )
            for d in (2, 1, 3):
                tgt = (me + d) % N_DEV
                rdma = pltpu.make_async_remote_copy(
                    src_ref=partial_ref.at[pl.ds(tgt * ch, ch),
                                           pl.ds(h * nh, nh)],
                    dst_ref=rs_buf.at[h, d - 1],
                    send_sem=rs_send.at[h, d - 1],
                    recv_sem=rs_recv.at[h, d - 1],
                    device_id=(tgt,),
                    device_id_type=pl.DeviceIdType.MESH,
                )
                rdma.start()
                rs[(h, d)] = rdma

        ag = []
        local_copies = []
        for h in range(N_HALF):
            for d in (1, 3, 2):
                rs[(h, d)].wait_recv()
            acc_ref[:, pl.ds(h * nh, nh)] = (
                partial_ref[pl.ds(me * ch, ch), pl.ds(h * nh, nh)]
                + rs_buf[h, 0] + rs_buf[h, 1] + rs_buf[h, 2]
            )
            for d in (2, 1, 3):
                tgt = (me + d) % N_DEV
                rdma = pltpu.make_async_remote_copy(
                    src_ref=acc_ref.at[:, pl.ds(h * nh, nh)],
                    dst_ref=out_ref.at[pl.ds(me * ch, ch),
                                       pl.ds(h * nh, nh)],
                    send_sem=ag_send.at[h, d - 1],
                    recv_sem=ag_recv.at[h, d - 1],
                    device_id=(tgt,),
                    device_id_type=pl.DeviceIdType.MESH,
                )
                rdma.start()
                ag.append(rdma)
            cp = pltpu.make_async_copy(
                acc_ref.at[:, pl.ds(h * nh, nh)],
                out_ref.at[pl.ds(me * ch, ch), pl.ds(h * nh, nh)],
                local_sem.at[h],
            )
            cp.start()
            local_copies.append(cp)

        for r in rs.values():
            r.wait_send()
        for cp in local_copies:
            cp.wait()
        for r in ag:
            r.wait()

    return pl.pallas_call(
        body,
        out_shape=jax.ShapeDtypeStruct((m, n), jnp.float32),
        in_specs=[
            pl.BlockSpec(memory_space=pltpu.VMEM),
            pl.BlockSpec(memory_space=pltpu.VMEM),
        ],
        out_specs=pl.BlockSpec(memory_space=pltpu.MemorySpace.HBM),
        scratch_shapes=[
            pltpu.VMEM((m, n), jnp.float32),
            pltpu.VMEM((N_HALF, N_DEV - 1, ch, nh), jnp.float32),
            pltpu.VMEM((ch, n), jnp.float32),
            pltpu.SemaphoreType.DMA((N_HALF, N_DEV - 1)),
            pltpu.SemaphoreType.DMA((N_HALF, N_DEV - 1)),
            pltpu.SemaphoreType.DMA((N_HALF, N_DEV - 1)),
            pltpu.SemaphoreType.DMA((N_HALF, N_DEV - 1)),
            pltpu.SemaphoreType.DMA((N_HALF,)),
        ],
        compiler_params=pltpu.CompilerParams(collective_id=0),
    )(A, B)


# device time: 14875 ns/iter; 1.3434x vs baseline; 1.3308x over previous
import jax
import jax.numpy as jnp
from jax import lax
from jax.experimental import pallas as pl
from jax.experimental.pallas import tpu as pltpu

N_DEV = 4
N_HALF = 2


def kernel(A, B):
    m, k = A.shape
    _, n = B.shape
    ch = m // N_DEV
    nh = n // N_HALF

    def body(a_ref, b_ref, out_ref, partial_ref, rs_buf, ag_buf,
             rs_send, rs_recv, ag_send, ag_recv):
        me = lax.axis_index("i")

        bar = pltpu.get_barrier_semaphore()
        for d in range(1, N_DEV):
            pl.semaphore_signal(
                bar, inc=1,
                device_id=((me + d) % N_DEV,),
                device_id_type=pl.DeviceIdType.MESH,
            )
        pl.semaphore_wait(bar, N_DEV - 1)

        rs = {}
        for h in range(N_HALF):
            partial_ref[:, pl.ds(h * nh, nh)] = jnp.dot(
                a_ref[:, :], b_ref[:, pl.ds(h * nh, nh)],
                preferred_element_type=jnp.float32,
            ).astype(jnp.bfloat16)
            for d in (2, 1, 3):
                tgt = (me + d) % N_DEV
                rdma = pltpu.make_async_remote_copy(
                    src_ref=partial_ref.at[pl.ds(tgt * ch, ch),
                                           pl.ds(h * nh, nh)],
                    dst_ref=rs_buf.at[h, d - 1],
                    send_sem=rs_send.at[h, d - 1],
                    recv_sem=rs_recv.at[h, d - 1],
                    device_id=(tgt,),
                    device_id_type=pl.DeviceIdType.MESH,
                )
                rdma.start()
                rs[(h, d)] = rdma

        ag = {}
        for h in range(N_HALF):
            for d in (1, 3, 2):
                rs[(h, d)].wait_recv()
            acc = (
                partial_ref[pl.ds(me * ch, ch),
                            pl.ds(h * nh, nh)].astype(jnp.float32)
                + rs_buf[h, 0].astype(jnp.float32)
                + rs_buf[h, 1].astype(jnp.float32)
                + rs_buf[h, 2].astype(jnp.float32)
            )
            out_ref[pl.ds(me * ch, ch), pl.ds(h * nh, nh)] = acc
            partial_ref[pl.ds(me * ch, ch), pl.ds(h * nh, nh)] = (
                acc.astype(jnp.bfloat16)
            )
            for d in (2, 1, 3):
                tgt = (me + d) % N_DEV
                rdma = pltpu.make_async_remote_copy(
                    src_ref=partial_ref.at[pl.ds(me * ch, ch),
                                           pl.ds(h * nh, nh)],
                    dst_ref=ag_buf.at[h, d - 1],
                    send_sem=ag_send.at[h, d - 1],
                    recv_sem=ag_recv.at[h, d - 1],
                    device_id=(tgt,),
                    device_id_type=pl.DeviceIdType.MESH,
                )
                rdma.start()
                ag[(h, d)] = rdma

        for h in range(N_HALF):
            for d in (1, 3, 2):
                ag[(h, d)].wait_recv()
                src = (me - d) % N_DEV
                out_ref[pl.ds(src * ch, ch), pl.ds(h * nh, nh)] = (
                    ag_buf[h, d - 1].astype(jnp.float32)
                )

        for r in rs.values():
            r.wait_send()
        for r in ag.values():
            r.wait_send()

    return pl.pallas_call(
        body,
        out_shape=jax.ShapeDtypeStruct((m, n), jnp.float32),
        in_specs=[
            pl.BlockSpec(memory_space=pltpu.VMEM),
            pl.BlockSpec(memory_space=pltpu.VMEM),
        ],
        out_specs=pl.BlockSpec(memory_space=pltpu.VMEM),
        scratch_shapes=[
            pltpu.VMEM((m, n), jnp.bfloat16),
            pltpu.VMEM((N_HALF, N_DEV - 1, ch, nh), jnp.bfloat16),
            pltpu.VMEM((N_HALF, N_DEV - 1, ch, nh), jnp.bfloat16),
            pltpu.SemaphoreType.DMA((N_HALF, N_DEV - 1)),
            pltpu.SemaphoreType.DMA((N_HALF, N_DEV - 1)),
            pltpu.SemaphoreType.DMA((N_HALF, N_DEV - 1)),
            pltpu.SemaphoreType.DMA((N_HALF, N_DEV - 1)),
        ],
        compiler_params=pltpu.CompilerParams(collective_id=0),
    )(A, B)


# device time: 14652 ns/iter; 1.3638x vs baseline; 1.0152x over previous
import jax
import jax.numpy as jnp
from jax import lax
from jax.experimental import pallas as pl
from jax.experimental.pallas import tpu as pltpu

N_DEV = 4
N_HALF = 4


def kernel(A, B):
    m, k = A.shape
    _, n = B.shape
    ch = m // N_DEV
    nh = n // N_HALF

    def body(a_ref, b_ref, out_ref, partial_ref, rs_buf, ag_buf,
             rs_send, rs_recv, ag_send, ag_recv):
        me = lax.axis_index("i")

        bar = pltpu.get_barrier_semaphore()
        for d in range(1, N_DEV):
            pl.semaphore_signal(
                bar, inc=1,
                device_id=((me + d) % N_DEV,),
                device_id_type=pl.DeviceIdType.MESH,
            )
        pl.semaphore_wait(bar, N_DEV - 1)

        rs = {}
        for h in range(N_HALF):
            partial_ref[:, pl.ds(h * nh, nh)] = jnp.dot(
                a_ref[:, :], b_ref[:, pl.ds(h * nh, nh)],
                preferred_element_type=jnp.float32,
            ).astype(jnp.bfloat16)
            for d in (2, 1, 3):
                tgt = (me + d) % N_DEV
                rdma = pltpu.make_async_remote_copy(
                    src_ref=partial_ref.at[pl.ds(tgt * ch, ch),
                                           pl.ds(h * nh, nh)],
                    dst_ref=rs_buf.at[h, d - 1],
                    send_sem=rs_send.at[h, d - 1],
                    recv_sem=rs_recv.at[h, d - 1],
                    device_id=(tgt,),
                    device_id_type=pl.DeviceIdType.MESH,
                )
                rdma.start()
                rs[(h, d)] = rdma

        ag = {}
        for h in range(N_HALF):
            for d in (1, 3, 2):
                rs[(h, d)].wait_recv()
            acc = (
                partial_ref[pl.ds(me * ch, ch),
                            pl.ds(h * nh, nh)].astype(jnp.float32)
                + rs_buf[h, 0].astype(jnp.float32)
                + rs_buf[h, 1].astype(jnp.float32)
                + rs_buf[h, 2].astype(jnp.float32)
            )
            out_ref[pl.ds(me * ch, ch), pl.ds(h * nh, nh)] = acc
            partial_ref[pl.ds(me * ch, ch), pl.ds(h * nh, nh)] = (
                acc.astype(jnp.bfloat16)
            )
            for d in (2, 1, 3):
                tgt = (me + d) % N_DEV
                rdma = pltpu.make_async_remote_copy(
                    src_ref=partial_ref.at[pl.ds(me * ch, ch),
                                           pl.ds(h * nh, nh)],
                    dst_ref=ag_buf.at[h, d - 1],
                    send_sem=ag_send.at[h, d - 1],
                    recv_sem=ag_recv.at[h, d - 1],
                    device_id=(tgt,),
                    device_id_type=pl.DeviceIdType.MESH,
                )
                rdma.start()
                ag[(h, d)] = rdma

        for h in range(N_HALF):
            for d in (1, 3, 2):
                ag[(h, d)].wait_recv()
                src = (me - d) % N_DEV
                out_ref[pl.ds(src * ch, ch), pl.ds(h * nh, nh)] = (
                    ag_buf[h, d - 1].astype(jnp.float32)
                )

        for r in rs.values():
            r.wait_send()
        for r in ag.values():
            r.wait_send()

    return pl.pallas_call(
        body,
        out_shape=jax.ShapeDtypeStruct((m, n), jnp.float32),
        in_specs=[
            pl.BlockSpec(memory_space=pltpu.VMEM),
            pl.BlockSpec(memory_space=pltpu.VMEM),
        ],
        out_specs=pl.BlockSpec(memory_space=pltpu.VMEM),
        scratch_shapes=[
            pltpu.VMEM((m, n), jnp.bfloat16),
            pltpu.VMEM((N_HALF, N_DEV - 1, ch, nh), jnp.bfloat16),
            pltpu.VMEM((N_HALF, N_DEV - 1, ch, nh), jnp.bfloat16),
            pltpu.SemaphoreType.DMA((N_HALF, N_DEV - 1)),
            pltpu.SemaphoreType.DMA((N_HALF, N_DEV - 1)),
            pltpu.SemaphoreType.DMA((N_HALF, N_DEV - 1)),
            pltpu.SemaphoreType.DMA((N_HALF, N_DEV - 1)),
        ],
        compiler_params=pltpu.CompilerParams(collective_id=0),
    )(A, B)


# device time: 14373 ns/iter; 1.3903x vs baseline; 1.0194x over previous
import jax
import jax.numpy as jnp
from jax import lax
from jax.experimental import pallas as pl
from jax.experimental.pallas import tpu as pltpu

N_DEV = 4
N_HALF = 4


def kernel(A, B):
    m, k = A.shape
    _, n = B.shape
    ch = m // N_DEV
    nh = n // N_HALF

    def body(a_ref, b_ref, out_ref, partial_ref, rs_buf, ag_buf,
             rs_send, rs_recv, ag_send, ag_recv):
        me = lax.axis_index("i")

        bar = pltpu.get_barrier_semaphore()
        for d in range(1, N_DEV):
            pl.semaphore_signal(
                bar, inc=1,
                device_id=((me + d) % N_DEV,),
                device_id_type=pl.DeviceIdType.MESH,
            )

        rs = {}
        for h in range(N_HALF):
            partial_ref[:, pl.ds(h * nh, nh)] = jnp.dot(
                a_ref[:, :], b_ref[:, pl.ds(h * nh, nh)],
                preferred_element_type=jnp.float32,
            ).astype(jnp.bfloat16)
            if h == 0:
                pl.semaphore_wait(bar, N_DEV - 1)
            for d in (2, 1, 3):
                tgt = (me + d) % N_DEV
                rdma = pltpu.make_async_remote_copy(
                    src_ref=partial_ref.at[pl.ds(tgt * ch, ch),
                                           pl.ds(h * nh, nh)],
                    dst_ref=rs_buf.at[h, d - 1],
                    send_sem=rs_send.at[h, d - 1],
                    recv_sem=rs_recv.at[h, d - 1],
                    device_id=(tgt,),
                    device_id_type=pl.DeviceIdType.MESH,
                )
                rdma.start()
                rs[(h, d)] = rdma

        ag = {}
        for h in range(N_HALF):
            for d in (1, 3, 2):
                rs[(h, d)].wait_recv()
            acc = (
                partial_ref[pl.ds(me * ch, ch),
                            pl.ds(h * nh, nh)].astype(jnp.float32)
                + rs_buf[h, 0].astype(jnp.float32)
                + rs_buf[h, 1].astype(jnp.float32)
                + rs_buf[h, 2].astype(jnp.float32)
            )
            out_ref[pl.ds(me * ch, ch), pl.ds(h * nh, nh)] = acc
            partial_ref[pl.ds(me * ch, ch), pl.ds(h * nh, nh)] = (
                acc.astype(jnp.bfloat16)
            )
            for d in (2, 1, 3):
                tgt = (me + d) % N_DEV
                rdma = pltpu.make_async_remote_copy(
                    src_ref=partial_ref.at[pl.ds(me * ch, ch),
                                           pl.ds(h * nh, nh)],
                    dst_ref=ag_buf.at[h, d - 1],
                    send_sem=ag_send.at[h, d - 1],
                    recv_sem=ag_recv.at[h, d - 1],
                    device_id=(tgt,),
                    device_id_type=pl.DeviceIdType.MESH,
                )
                rdma.start()
                ag[(h, d)] = rdma

        for h in range(N_HALF):
            for d in (1, 3, 2):
                ag[(h, d)].wait_recv()
                src = (me - d) % N_DEV
                out_ref[pl.ds(src * ch, ch), pl.ds(h * nh, nh)] = (
                    ag_buf[h, d - 1].astype(jnp.float32)
                )

        for r in rs.values():
            r.wait_send()
        for r in ag.values():
            r.wait_send()

    return pl.pallas_call(
        body,
        out_shape=jax.ShapeDtypeStruct((m, n), jnp.float32),
        in_specs=[
            pl.BlockSpec(memory_space=pltpu.VMEM),
            pl.BlockSpec(memory_space=pltpu.VMEM),
        ],
        out_specs=pl.BlockSpec(memory_space=pltpu.VMEM),
        scratch_shapes=[
            pltpu.VMEM((m, n), jnp.bfloat16),
            pltpu.VMEM((N_HALF, N_DEV - 1, ch, nh), jnp.bfloat16),
            pltpu.VMEM((N_HALF, N_DEV - 1, ch, nh), jnp.bfloat16),
            pltpu.SemaphoreType.DMA((N_HALF, N_DEV - 1)),
            pltpu.SemaphoreType.DMA((N_HALF, N_DEV - 1)),
            pltpu.SemaphoreType.DMA((N_HALF, N_DEV - 1)),
            pltpu.SemaphoreType.DMA((N_HALF, N_DEV - 1)),
        ],
        compiler_params=pltpu.CompilerParams(collective_id=0),
    )(A, B)


# device time: 14371 ns/iter; 1.3905x vs baseline; 1.0001x over previous
import jax
import jax.numpy as jnp
from jax import lax
from jax.experimental import pallas as pl
from jax.experimental.pallas import tpu as pltpu

N_DEV = 4
N_HALF = 4


def kernel(A, B):
    m, k = A.shape
    _, n = B.shape
    ch = m // N_DEV
    nh = n // N_HALF

    def body(a_ref, b_ref, out_ref, partial_ref, rs_buf, ag_buf,
             rs_send, rs_recv, ag_send, ag_recv):
        me = lax.axis_index("i")

        bar = pltpu.get_barrier_semaphore()
        for d in range(1, N_DEV):
            pl.semaphore_signal(
                bar, inc=1,
                device_id=((me + d) % N_DEV,),
                device_id_type=pl.DeviceIdType.MESH,
            )

        rs = {}
        for h in range(N_HALF):
            partial_ref[:, pl.ds(h * nh, nh)] = jnp.dot(
                a_ref[:, :], b_ref[:, pl.ds(h * nh, nh)],
                preferred_element_type=jnp.float32,
            ).astype(jnp.bfloat16)
            if h == 0:
                pl.semaphore_wait(bar, N_DEV - 1)
            for d in (2, 1, 3):
                tgt = (me + d) % N_DEV
                rdma = pltpu.make_async_remote_copy(
                    src_ref=partial_ref.at[pl.ds(tgt * ch, ch),
                                           pl.ds(h * nh, nh)],
                    dst_ref=rs_buf.at[h, d - 1],
                    send_sem=rs_send.at[h, d - 1],
                    recv_sem=rs_recv.at[h, d - 1],
                    device_id=(tgt,),
                    device_id_type=pl.DeviceIdType.MESH,
                )
                rdma.start()
                rs[(h, d)] = rdma

        ag = {}
        for h in range(N_HALF):
            for d in (1, 3, 2):
                rs[(h, d)].wait_recv()
            acc = (
                partial_ref[pl.ds(me * ch, ch),
                            pl.ds(h * nh, nh)].astype(jnp.float32)
                + rs_buf[h, 0].astype(jnp.float32)
                + rs_buf[h, 1].astype(jnp.float32)
                + rs_buf[h, 2].astype(jnp.float32)
            )
            partial_ref[pl.ds(me * ch, ch), pl.ds(h * nh, nh)] = (
                acc.astype(jnp.bfloat16)
            )
            for d in (2, 1, 3):
                tgt = (me + d) % N_DEV
                rdma = pltpu.make_async_remote_copy(
                    src_ref=partial_ref.at[pl.ds(me * ch, ch),
                                           pl.ds(h * nh, nh)],
                    dst_ref=ag_buf.at[h, d - 1],
                    send_sem=ag_send.at[h, d - 1],
                    recv_sem=ag_recv.at[h, d - 1],
                    device_id=(tgt,),
                    device_id_type=pl.DeviceIdType.MESH,
                )
                rdma.start()
                ag[(h, d)] = rdma
            out_ref[pl.ds(me * ch, ch), pl.ds(h * nh, nh)] = acc

        for h in range(N_HALF):
            for d in (1, 3, 2):
                ag[(h, d)].wait_recv()
                src = (me - d) % N_DEV
                out_ref[pl.ds(src * ch, ch), pl.ds(h * nh, nh)] = (
                    ag_buf[h, d - 1].astype(jnp.float32)
                )

        for r in rs.values():
            r.wait_send()
        for r in ag.values():
            r.wait_send()

    return pl.pallas_call(
        body,
        out_shape=jax.ShapeDtypeStruct((m, n), jnp.float32),
        in_specs=[
            pl.BlockSpec(memory_space=pltpu.VMEM),
            pl.BlockSpec(memory_space=pltpu.VMEM),
        ],
        out_specs=pl.BlockSpec(memory_space=pltpu.VMEM),
        scratch_shapes=[
            pltpu.VMEM((m, n), jnp.bfloat16),
            pltpu.VMEM((N_HALF, N_DEV - 1, ch, nh), jnp.bfloat16),
            pltpu.VMEM((N_HALF, N_DEV - 1, ch, nh), jnp.bfloat16),
            pltpu.SemaphoreType.DMA((N_HALF, N_DEV - 1)),
            pltpu.SemaphoreType.DMA((N_HALF, N_DEV - 1)),
            pltpu.SemaphoreType.DMA((N_HALF, N_DEV - 1)),
            pltpu.SemaphoreType.DMA((N_HALF, N_DEV - 1)),
        ],
        compiler_params=pltpu.CompilerParams(collective_id=0),
    )(A, B)
